# Initial kernel scaffold; baseline (speedup 1.0000x reference)
import jax
import jax.numpy as jnp
from jax.experimental import pallas as pl


def kernel(input, weight, absmax, code):
    raise NotImplementedError("write your pallas kernel here")


# speedup vs baseline: 1026.6045x; 1026.6045x over previous
import functools

import jax
import jax.numpy as jnp
from jax import lax
from jax.experimental import pallas as pl
from jax.experimental.pallas import tpu as pltpu
from jax.experimental.pallas import tpu_sc as plsc

_BLOCK = 4096
_DIM = 64
_L = 16
_GC = 128


def _make_kernel(batch, num_emb, n_blocks_padded):
    info = plsc.get_sparse_core_info()
    nw = info.num_cores * info.num_subcores
    b_per_w = batch // nw
    nch = b_per_w // _GC
    mesh = plsc.VectorSubcoreMesh(core_axis_name="c", subcore_axis_name="s")

    @functools.partial(
        pl.kernel,
        mesh=mesh,
        out_type=jax.ShapeDtypeStruct((batch, _DIM), jnp.float32),
        scratch_types=[
            pltpu.VMEM((nch, _GC), jnp.int32),
            pltpu.VMEM((b_per_w, _DIM), jnp.int32),
            pltpu.VMEM((256,), jnp.float32),
            pltpu.VMEM((n_blocks_padded,), jnp.float32),
            pltpu.VMEM((b_per_w,), jnp.float32),
            pltpu.VMEM((b_per_w, _DIM), jnp.float32),
            pltpu.SemaphoreType.DMA,
        ],
        compiler_params=pltpu.CompilerParams(
            needs_layout_passes=False, use_tc_tiling_on_sc=False),
    )
    def k(idx_hbm, w_hbm, amax_hbm, code_hbm, out_hbm,
          idx_v, rows_v, code_v, amax_v, arow_v, out_v, sem):
        wid = lax.axis_index("s") * info.num_cores + lax.axis_index("c")
        base = wid * b_per_w

        for g in range(nch):
            pltpu.sync_copy(idx_hbm.at[pl.ds(base + g * _GC, _GC)],
                            idx_v.at[g])
        pltpu.sync_copy(code_hbm, code_v)
        pltpu.sync_copy(amax_hbm, amax_v)

        copies = []
        for g in range(nch):
            copies.append(
                pltpu.async_copy(w_hbm.at[idx_v.at[g]],
                                 rows_v.at[pl.ds(g * _GC, _GC)], sem))

        for g in range(nch):
            for j in range(_GC // _L):
                iv = idx_v[g, pl.ds(j * _L, _L)]
                blk = jax.lax.shift_right_logical(iv, 6)
                av = plsc.load_gather(amax_v, [blk])
                arow_v[pl.ds(g * _GC + j * _L, _L)] = av

        for c in copies:
            c.wait()

        def grp_body(j, carry):
            row0 = j * _L
            av = arow_v[pl.ds(row0, _L)]
            for r in range(_L):
                a = av[r]
                for kk in range(_DIM // _L):
                    cw = rows_v[row0 + r, pl.ds(kk * _L, _L)]
                    out_v[row0 + r, pl.ds(kk * _L, _L)] = (
                        plsc.load_gather(code_v, [cw]) * a)
            return carry

        lax.fori_loop(0, b_per_w // _L, grp_body, 0)

        pltpu.sync_copy(out_v, out_hbm.at[pl.ds(base, b_per_w)])

    return k


def kernel(input, weight, absmax, code):
    n_blocks = absmax.shape[0]
    pad = (-n_blocks) % _L
    amax_padded = jnp.concatenate(
        [absmax, jnp.zeros((pad,), absmax.dtype)]) if pad else absmax
    k = _make_kernel(input.shape[0], weight.shape[0], n_blocks + pad)
    return k(input, weight, amax_padded, code)


# speedup vs baseline: 1675.4680x; 1.6320x over previous
import functools

import jax
import jax.numpy as jnp
from jax import lax
from jax.experimental import pallas as pl
from jax.experimental.pallas import tpu as pltpu
from jax.experimental.pallas import tpu_sc as plsc

_DIM = 64
_L = 16


def _make_kernel(batch, n_blocks_padded):
    info = plsc.get_sparse_core_info()
    nw = info.num_cores * info.num_subcores
    b_per_w = batch // nw
    mesh = plsc.VectorSubcoreMesh(core_axis_name="c", subcore_axis_name="s")

    @functools.partial(
        pl.kernel,
        mesh=mesh,
        out_type=jax.ShapeDtypeStruct((batch * _DIM,), jnp.float32),
        scratch_types=[
            pltpu.VMEM((b_per_w,), jnp.int32),
            pltpu.VMEM((b_per_w, _DIM), jnp.int32),
            pltpu.VMEM((256,), jnp.float32),
            pltpu.VMEM((n_blocks_padded,), jnp.float32),
            pltpu.VMEM((b_per_w,), jnp.float32),
            pltpu.VMEM((b_per_w * _DIM,), jnp.float32),
            pltpu.SemaphoreType.DMA,
            pltpu.SemaphoreType.DMA,
        ],
        compiler_params=pltpu.CompilerParams(needs_layout_passes=False),
    )
    def k(idx_hbm, w_hbm, amax_hbm, code_hbm, out_hbm,
          idx_v, rows_v, code_v, amax_v, arow_v, out_v, sem, rsem):
        wid = lax.axis_index("s") * info.num_cores + lax.axis_index("c")
        base = wid * b_per_w

        pltpu.sync_copy(idx_hbm.at[pl.ds(base, b_per_w)], idx_v)
        pltpu.sync_copy(code_hbm, code_v)
        pltpu.sync_copy(amax_hbm, amax_v)

        def fire_body(j, carry):
            row0 = j * _L
            iv = idx_v[pl.ds(row0, _L)]
            for r in range(_L):
                pltpu.async_copy(
                    w_hbm.at[pl.ds(iv[r], 1)],
                    rows_v.at[pl.ds(row0 + r, 1)], rsem)
            blk = jax.lax.shift_right_logical(iv, 6)
            arow_v[pl.ds(row0, _L)] = plsc.load_gather(amax_v, [blk])
            return carry

        lax.fori_loop(0, b_per_w // _L, fire_body, 0)

        def drain_body(j, carry):
            pltpu.make_async_copy(
                w_hbm.at[pl.ds(0, 1)], rows_v.at[pl.ds(0, 1)], rsem).wait()
            return carry

        lax.fori_loop(0, b_per_w, drain_body, 0)

        def grp_body(j, carry):
            row0 = j * _L
            av = arow_v[pl.ds(row0, _L)]
            for r in range(_L):
                a = av[r]
                for kk in range(_DIM // _L):
                    cw = rows_v[row0 + r, pl.ds(kk * _L, _L)]
                    out_v[pl.ds((row0 + r) * _DIM + kk * _L, _L)] = (
                        plsc.load_gather(code_v, [cw]) * a)
            return carry

        lax.fori_loop(0, b_per_w // _L, grp_body, 0)

        pltpu.sync_copy(out_v, out_hbm.at[pl.ds(base * _DIM, b_per_w * _DIM)])

    return k


def kernel(input, weight, absmax, code):
    n_blocks = absmax.shape[0]
    pad = (-n_blocks) % _L
    amax_padded = jnp.concatenate(
        [absmax, jnp.zeros((pad,), absmax.dtype)]) if pad else absmax
    k = _make_kernel(input.shape[0], n_blocks + pad)
    out_flat = k(input, weight, amax_padded, code)
    return out_flat.reshape(input.shape[0], _DIM)
